# baseline (speedup 1.0000x reference)
import functools

import jax
import jax.numpy as jnp
from jax import lax
from jax.experimental import pallas as pl
from jax.experimental.pallas import tpu as pltpu

N = 10000
E = 320000
HEADS = 8
OUT = 16
D = 128
BN = 1000


def _tc1_body(x_ref, ws_ref, wd_ref, wr_ref, al_ref, ar_ref, b_ref,
              fs_ref, el_ref, er_ref, res_ref):
    x = x_ref[...]
    fs = jnp.dot(x, ws_ref[...], preferred_element_type=jnp.float32)
    fd = jnp.dot(x, wd_ref[...], preferred_element_type=jnp.float32)
    fs_ref[...] = fs
    el = jnp.dot(fs, al_ref[...], preferred_element_type=jnp.float32)
    er = jnp.dot(fd, ar_ref[...], preferred_element_type=jnp.float32)
    el_ref[...] = jnp.concatenate(
        [el, jnp.full((BN, HEADS), -1e30, jnp.float32)], axis=1)
    er_ref[...] = jnp.concatenate(
        [er, jnp.zeros((BN, HEADS), jnp.float32)], axis=1)
    res_ref[...] = jnp.dot(x, wr_ref[...], preferred_element_type=jnp.float32) \
        + b_ref[...]


def _tc1(x, w_src, w_dst, w_res, a_l, a_r, bias):
    grid = N // BN
    wspec = pl.BlockSpec((D, D), lambda i: (0, 0))
    return pl.pallas_call(
        _tc1_body,
        grid=(grid,),
        in_specs=[
            pl.BlockSpec((BN, D), lambda i: (i, 0)),
            wspec, wspec, wspec,
            pl.BlockSpec((D, HEADS), lambda i: (0, 0)),
            pl.BlockSpec((D, HEADS), lambda i: (0, 0)),
            pl.BlockSpec((1, D), lambda i: (0, 0)),
        ],
        out_specs=[
            pl.BlockSpec((BN, D), lambda i: (i, 0)),
            pl.BlockSpec((BN, OUT), lambda i: (i, 0)),
            pl.BlockSpec((BN, OUT), lambda i: (i, 0)),
            pl.BlockSpec((BN, D), lambda i: (i, 0)),
        ],
        out_shape=[
            jax.ShapeDtypeStruct((N, D), jnp.float32),
            jax.ShapeDtypeStruct((N, OUT), jnp.float32),
            jax.ShapeDtypeStruct((N, OUT), jnp.float32),
            jax.ShapeDtypeStruct((N, D), jnp.float32),
        ],
    )(x, w_src, w_dst, w_res, a_l, a_r, bias)


def _tc2_body(acca_ref, accb_ref, dena_ref, denb_ref, res_ref, b16_ref, o_ref):
    den = dena_ref[...] + denb_ref[...]
    denw = jnp.dot(den, b16_ref[...],
                   preferred_element_type=jnp.float32)
    safe = jnp.where(denw > 0.0, denw, 1.0)
    y = (acca_ref[...] + accb_ref[...]) / safe + res_ref[...]
    o_ref[...] = jnp.where(y > 0.0, y, jnp.exp(jnp.minimum(y, 0.0)) - 1.0)


def _tc2(acc_a, acc_b, den_a, den_b, res, b16):
    grid = N // BN
    return pl.pallas_call(
        _tc2_body,
        grid=(grid,),
        in_specs=[
            pl.BlockSpec((BN, D), lambda i: (i, 0)),
            pl.BlockSpec((BN, D), lambda i: (i, 0)),
            pl.BlockSpec((BN, OUT), lambda i: (i, 0)),
            pl.BlockSpec((BN, OUT), lambda i: (i, 0)),
            pl.BlockSpec((BN, D), lambda i: (i, 0)),
            pl.BlockSpec((OUT, D), lambda i: (0, 0)),
        ],
        out_specs=pl.BlockSpec((BN, D), lambda i: (i, 0)),
        out_shape=jax.ShapeDtypeStruct((N, D), jnp.float32),
    )(acc_a, acc_b, den_a, den_b, res, b16)


def _edge_xla(fsrc, el16, er16, src, dst):
    e = el16[src, :HEADS] + er16[dst, :HEADS]
    ee = jnp.exp(jnp.where(e >= 0.0, e, 0.2 * e))
    den = jax.ops.segment_sum(ee, dst, num_segments=N)
    den16 = jnp.concatenate([den, jnp.zeros((N, HEADS), jnp.float32)], axis=1)
    eew = jnp.repeat(ee, OUT, axis=1)
    acc = jax.ops.segment_sum(eew * fsrc[src], dst, num_segments=N)
    return acc, den16


def _expand_attn(a):
    eye = jnp.eye(HEADS, dtype=jnp.float32)
    return (a[:, :, None] * eye[:, None, :]).reshape(HEADS * OUT, HEADS)


def kernel(h_emo, h_cau, h_pair, edge_index_emo, edge_index_cau,
           edge_index_pair, doc_len, params):
    feats = (h_emo, h_cau, h_pair)
    eis = (edge_index_emo, edge_index_cau, edge_index_pair)
    b16 = (jnp.eye(HEADS, dtype=jnp.float32)[:, :, None]
           * jnp.ones((1, 1, OUT), jnp.float32)).reshape(HEADS, D)
    b16 = jnp.concatenate([b16, jnp.zeros((HEADS, D), jnp.float32)], axis=0)

    outs = []
    for i in range(3):
        x = feats[i]
        ei = eis[i]
        p = params
        a_l = _expand_attn(p['attn_l_%d' % i])
        a_r = _expand_attn(p['attn_r_%d' % i])
        bias = p['bias_%d' % i].reshape(1, D)
        fsrc, el16, er16, res = _tc1(
            x, p['W_src_%d' % i], p['W_dst_%d' % i], p['W_res_%d' % i],
            a_l, a_r, bias)
        acc, den16 = _edge_xla(fsrc, el16, er16, ei[0], ei[1])
        zacc = jnp.zeros((N, D), jnp.float32)
        zden = jnp.zeros((N, OUT), jnp.float32)
        outs.append(_tc2(acc, zacc, den16, zden, res, b16))
    return tuple(outs)


# speedup vs baseline: 161.4057x; 161.4057x over previous
import functools

import jax
import jax.numpy as jnp
from jax import lax
from jax.experimental import pallas as pl
from jax.experimental.pallas import tpu as pltpu
from jax.experimental.pallas import tpu_sc as plsc

N = 10000
E = 320000
HEADS = 8
OUT = 16
D = 128
BN = 1000

NC = 2
NS = 16
NW = NC * NS
EPW = E // NW
CH = 128
NCHUNK = -(-EPW // CH)
EPAD = NCHUNK * CH
NPAD = 10112
NPT = NPAD // NS


def _tc1_body(x_ref, ws_ref, wd_ref, wr_ref, al_ref, ar_ref, b_ref,
              fs_ref, el_ref, er_ref, res_ref):
    x = x_ref[...]
    fs = jnp.dot(x, ws_ref[...], preferred_element_type=jnp.float32)
    fd = jnp.dot(x, wd_ref[...], preferred_element_type=jnp.float32)
    fs_ref[...] = fs
    el = jnp.dot(fs, al_ref[...], preferred_element_type=jnp.float32)
    er = jnp.dot(fd, ar_ref[...], preferred_element_type=jnp.float32)
    el_ref[...] = jnp.concatenate(
        [el, jnp.full((BN, HEADS), -1e30, jnp.float32)], axis=1)
    er_ref[...] = jnp.concatenate(
        [er, jnp.zeros((BN, HEADS), jnp.float32)], axis=1)
    res_ref[...] = jnp.dot(x, wr_ref[...], preferred_element_type=jnp.float32) \
        + b_ref[...]


def _tc1(x, w_src, w_dst, w_res, a_l, a_r, bias):
    grid = N // BN
    wspec = pl.BlockSpec((D, D), lambda i: (0, 0))
    return pl.pallas_call(
        _tc1_body,
        grid=(grid,),
        in_specs=[
            pl.BlockSpec((BN, D), lambda i: (i, 0)),
            wspec, wspec, wspec,
            pl.BlockSpec((D, HEADS), lambda i: (0, 0)),
            pl.BlockSpec((D, HEADS), lambda i: (0, 0)),
            pl.BlockSpec((1, D), lambda i: (0, 0)),
        ],
        out_specs=[
            pl.BlockSpec((BN, D), lambda i: (i, 0)),
            pl.BlockSpec((BN, OUT), lambda i: (i, 0)),
            pl.BlockSpec((BN, OUT), lambda i: (i, 0)),
            pl.BlockSpec((BN, D), lambda i: (i, 0)),
        ],
        out_shape=[
            jax.ShapeDtypeStruct((N, D), jnp.float32),
            jax.ShapeDtypeStruct((N, OUT), jnp.float32),
            jax.ShapeDtypeStruct((N, OUT), jnp.float32),
            jax.ShapeDtypeStruct((N, D), jnp.float32),
        ],
    )(x, w_src, w_dst, w_res, a_l, a_r, bias)


def _tc2_body(acca_ref, accb_ref, dena_ref, denb_ref, res_ref, b16_ref, o_ref):
    den = dena_ref[...] + denb_ref[...]
    denw = jnp.dot(den, b16_ref[...],
                   preferred_element_type=jnp.float32)
    safe = jnp.where(denw > 0.0, denw, 1.0)
    y = (acca_ref[...] + accb_ref[...]) / safe + res_ref[...]
    o_ref[...] = jnp.where(y > 0.0, y, jnp.exp(jnp.minimum(y, 0.0)) - 1.0)


def _tc2(acc_a, acc_b, den_a, den_b, res, b16):
    grid = N // BN
    return pl.pallas_call(
        _tc2_body,
        grid=(grid,),
        in_specs=[
            pl.BlockSpec((BN, D), lambda i: (i, 0)),
            pl.BlockSpec((BN, D), lambda i: (i, 0)),
            pl.BlockSpec((BN, HEADS), lambda i: (i, 0)),
            pl.BlockSpec((BN, HEADS), lambda i: (i, 0)),
            pl.BlockSpec((BN, D), lambda i: (i, 0)),
            pl.BlockSpec((HEADS, D), lambda i: (0, 0)),
        ],
        out_specs=pl.BlockSpec((BN, D), lambda i: (i, 0)),
        out_shape=jax.ShapeDtypeStruct((N, D), jnp.float32),
    )(acc_a, acc_b, den_a, den_b, res, b16)


def _sc_edge_body(*refs):
    (f0, f1, f2, l0, l1, l2, r0_, r1_, r2_, s0, s1, s2, d0, d1, d2,
     zacc_h, zden_h,
     a0, a1, a2, n0, n1, n2,
     src_v, dst_v, rows_v, elv, erv, eev, acc_sp, den_sp, sem) = refs
    c = lax.axis_index("c")
    s = lax.axis_index("s")
    wid = c * NS + s
    r0 = s * NPT

    for (fsrc_h, el_h, er_h, src_h, dst_h, acc_o, den_o) in (
            (f0, l0, r0_, s0, d0, a0, n0),
            (f1, l1, r1_, s1, d1, a1, n1),
            (f2, l2, r2_, s2, d2, a2, n2)):
        pltpu.sync_copy(zacc_h.at[pl.ds(r0, NPT)], acc_sp.at[pl.ds(r0, NPT)])
        pltpu.sync_copy(zden_h.at[pl.ds(r0, NPT)], den_sp.at[pl.ds(r0, NPT)])
        pltpu.sync_copy(src_h.at[wid], src_v)
        pltpu.sync_copy(dst_h.at[wid], dst_v)
        plsc.subcore_barrier()

        _LANE = lax.iota(jnp.int32, OUT)
        _LANEMASK = _LANE < HEADS

        def chunk(j, carry):
            sidx = src_v.at[j]
            didx = dst_v.at[j]
            pltpu.async_copy(fsrc_h.at[sidx], rows_v, sem).wait()
            pltpu.async_copy(el_h.at[sidx], elv, sem).wait()
            pltpu.async_copy(er_h.at[didx], erv, sem).wait()
            base = j * CH

            def edge(e, carry2):
                x = elv[e, :] + erv[e, :]
                x = jnp.where(x >= 0.0, x, 0.2 * x)
                ee = jnp.exp(x)
                ee = jnp.where(base + e < EPW, ee,
                               jnp.zeros((OUT,), jnp.float32))
                plsc.store_scatter(
                    eev, [jnp.full((OUT,), e, jnp.int32), _LANE], ee,
                    mask=_LANEMASK)
                for h in range(HEADS):
                    w = jnp.take_along_axis(
                        ee, jnp.full((OUT,), h, jnp.int32), axis=0,
                        mode="promise_in_bounds")
                    rows_v[e, pl.ds(h * OUT, OUT)] = \
                        rows_v[e, pl.ds(h * OUT, OUT)] * w
                return carry2

            lax.fori_loop(0, CH, edge, 0)
            pltpu.sync_copy(eev, den_sp.at[didx], add=True)
            pltpu.sync_copy(rows_v, acc_sp.at[didx], add=True)
            return carry

        lax.fori_loop(0, NCHUNK, chunk, 0)
        plsc.subcore_barrier()
        pltpu.sync_copy(acc_sp.at[pl.ds(r0, NPT)], acc_o.at[c, pl.ds(r0, NPT)])
        pltpu.sync_copy(den_sp.at[pl.ds(r0, NPT)], den_o.at[c, pl.ds(r0, NPT)])
        plsc.subcore_barrier()


def _sc_edge(fsrcs, els, ers, srcs, dsts, zacc, zden):
    mesh = plsc.VectorSubcoreMesh(
        core_axis_name="c", subcore_axis_name="s",
        num_cores=NC, num_subcores=NS)
    f = pl.kernel(
        _sc_edge_body,
        out_type=[jax.ShapeDtypeStruct((NC, NPAD, D), jnp.float32)] * 3
        + [jax.ShapeDtypeStruct((NC, NPAD, HEADS), jnp.float32)] * 3,
        mesh=mesh,
        compiler_params=pltpu.CompilerParams(
            use_tc_tiling_on_sc=False, needs_layout_passes=False),
        scratch_types=[
            pltpu.VMEM((NCHUNK, CH), jnp.int32),
            pltpu.VMEM((NCHUNK, CH), jnp.int32),
            pltpu.VMEM((CH, D), jnp.float32),
            pltpu.VMEM((CH, OUT), jnp.float32),
            pltpu.VMEM((CH, OUT), jnp.float32),
            pltpu.VMEM((CH, HEADS), jnp.float32),
            pltpu.VMEM_SHARED((NPAD, D), jnp.float32),
            pltpu.VMEM_SHARED((NPAD, HEADS), jnp.float32),
            pltpu.SemaphoreType.DMA,
        ])
    return f(*fsrcs, *els, *ers, *srcs, *dsts, zacc, zden)


def _pad_edges(ei):
    pad = jnp.broadcast_to(
        (jnp.arange(EPAD - EPW, dtype=jnp.int32) % N)[None],
        (NW, EPAD - EPW))
    def prep(v):
        v2 = v.astype(jnp.int32).reshape(NW, EPW)
        return jnp.concatenate([v2, pad], axis=1).reshape(NW, NCHUNK, CH)
    return prep(ei[0]), prep(ei[1])


def _expand_attn(a):
    eye = jnp.eye(HEADS, dtype=jnp.float32)
    return (a[:, :, None] * eye[:, None, :]).reshape(HEADS * OUT, HEADS)


def kernel(h_emo, h_cau, h_pair, edge_index_emo, edge_index_cau,
           edge_index_pair, doc_len, params):
    feats = (h_emo, h_cau, h_pair)
    eis = (edge_index_emo, edge_index_cau, edge_index_pair)
    b16 = (jnp.eye(HEADS, dtype=jnp.float32)[:, :, None]
           * jnp.ones((1, 1, OUT), jnp.float32)).reshape(HEADS, D)

    fsrcs, els, ers, srcs, dsts, ress = [], [], [], [], [], []
    p = params
    for i in range(3):
        a_l = _expand_attn(p['attn_l_%d' % i])
        a_r = _expand_attn(p['attn_r_%d' % i])
        bias = p['bias_%d' % i].reshape(1, D)
        fsrc, el16, er16, res = _tc1(
            feats[i], p['W_src_%d' % i], p['W_dst_%d' % i], p['W_res_%d' % i],
            a_l, a_r, bias)
        src3, dst3 = _pad_edges(eis[i])
        fsrcs.append(fsrc); els.append(el16); ers.append(er16)
        srcs.append(src3); dsts.append(dst3); ress.append(res)
    zacc = jnp.zeros((NPAD, D), jnp.float32)
    zden = jnp.zeros((NPAD, HEADS), jnp.float32)
    a0, a1, a2, n0, n1, n2 = _sc_edge(fsrcs, els, ers, srcs, dsts, zacc, zden)
    outs = []
    for i, (acc2, den2) in enumerate(((a0, n0), (a1, n1), (a2, n2))):
        outs.append(_tc2(acc2[0, :N], acc2[1, :N], den2[0, :N], den2[1, :N],
                         ress[i], b16))
    return tuple(outs)


# speedup vs baseline: 250.1771x; 1.5500x over previous
import functools

import jax
import jax.numpy as jnp
from jax import lax
from jax.experimental import pallas as pl
from jax.experimental.pallas import tpu as pltpu
from jax.experimental.pallas import tpu_sc as plsc

N = 10000
E = 320000
HEADS = 8
OUT = 16
D = 128
BN = 1000

NC = 2
NS = 16
NW = NC * NS
EPW = E // NW
CH = 128
NCHUNK = 80
EPAD = NCHUNK * CH
NPAD = 10112
NPT = NPAD // NS


def _tc1_body(x_ref, ws_ref, wd_ref, wr_ref, al_ref, ar_ref, b_ref,
              fs_ref, el_ref, er_ref, res_ref):
    x = x_ref[...]
    fs = jnp.dot(x, ws_ref[...], preferred_element_type=jnp.float32)
    fd = jnp.dot(x, wd_ref[...], preferred_element_type=jnp.float32)
    fs_ref[...] = fs
    el = jnp.dot(fs, al_ref[...], preferred_element_type=jnp.float32)
    er = jnp.dot(fd, ar_ref[...], preferred_element_type=jnp.float32)
    el_ref[...] = jnp.concatenate(
        [el, jnp.full((BN, HEADS), -1e30, jnp.float32)], axis=1)
    er_ref[...] = jnp.concatenate(
        [er, jnp.zeros((BN, HEADS), jnp.float32)], axis=1)
    res_ref[...] = jnp.dot(x, wr_ref[...], preferred_element_type=jnp.float32) \
        + b_ref[...]


def _tc1(x, w_src, w_dst, w_res, a_l, a_r, bias):
    grid = N // BN
    wspec = pl.BlockSpec((D, D), lambda i: (0, 0))
    return pl.pallas_call(
        _tc1_body,
        grid=(grid,),
        in_specs=[
            pl.BlockSpec((BN, D), lambda i: (i, 0)),
            wspec, wspec, wspec,
            pl.BlockSpec((D, HEADS), lambda i: (0, 0)),
            pl.BlockSpec((D, HEADS), lambda i: (0, 0)),
            pl.BlockSpec((1, D), lambda i: (0, 0)),
        ],
        out_specs=[
            pl.BlockSpec((BN, D), lambda i: (i, 0)),
            pl.BlockSpec((BN, OUT), lambda i: (i, 0)),
            pl.BlockSpec((BN, OUT), lambda i: (i, 0)),
            pl.BlockSpec((BN, D), lambda i: (i, 0)),
        ],
        out_shape=[
            jax.ShapeDtypeStruct((N, D), jnp.float32),
            jax.ShapeDtypeStruct((N, OUT), jnp.float32),
            jax.ShapeDtypeStruct((N, OUT), jnp.float32),
            jax.ShapeDtypeStruct((N, D), jnp.float32),
        ],
    )(x, w_src, w_dst, w_res, a_l, a_r, bias)


def _tc2_body(acca_ref, accb_ref, dena_ref, denb_ref, res_ref, b16_ref, o_ref):
    den = dena_ref[...] + denb_ref[...]
    denw = jnp.dot(den, b16_ref[...],
                   preferred_element_type=jnp.float32)
    safe = jnp.where(denw > 0.0, denw, 1.0)
    y = (acca_ref[...] + accb_ref[...]) / safe + res_ref[...]
    o_ref[...] = jnp.where(y > 0.0, y, jnp.exp(jnp.minimum(y, 0.0)) - 1.0)


def _tc2(acc_a, acc_b, den_a, den_b, res, b16):
    grid = N // BN
    return pl.pallas_call(
        _tc2_body,
        grid=(grid,),
        in_specs=[
            pl.BlockSpec((BN, D), lambda i: (i, 0)),
            pl.BlockSpec((BN, D), lambda i: (i, 0)),
            pl.BlockSpec((BN, HEADS), lambda i: (i, 0)),
            pl.BlockSpec((BN, HEADS), lambda i: (i, 0)),
            pl.BlockSpec((BN, D), lambda i: (i, 0)),
            pl.BlockSpec((HEADS, D), lambda i: (0, 0)),
        ],
        out_specs=pl.BlockSpec((BN, D), lambda i: (i, 0)),
        out_shape=jax.ShapeDtypeStruct((N, D), jnp.float32),
    )(acc_a, acc_b, den_a, den_b, res, b16)


def _sc_edge_body(*refs):
    (f0, f1, f2, l0, l1, l2, r0_, r1_, r2_, s0, s1, s2, d0, d1, d2,
     zacc_h, zden_h,
     a0, a1, a2, n0, n1, n2,
     sidx_a, sidx_b, didx_a, didx_b,
     rows_a, rows_b, el_a, el_b, er_a, er_b, eev,
     acc_sp, den_sp, sem_a, sem_b, sem_i) = refs
    c = lax.axis_index("c")
    s = lax.axis_index("s")
    wid = c * NS + s
    r0 = s * NPT
    lane = lax.iota(jnp.int32, OUT)
    lanemask = lane < HEADS

    for (fsrc_h, el_h, er_h, src_h, dst_h, acc_o, den_o) in (
            (f0, l0, r0_, s0, d0, a0, n0),
            (f1, l1, r1_, s1, d1, a1, n1),
            (f2, l2, r2_, s2, d2, a2, n2)):
        pltpu.sync_copy(zacc_h.at[pl.ds(r0, NPT)], acc_sp.at[pl.ds(r0, NPT)])
        pltpu.sync_copy(zden_h.at[pl.ds(r0, NPT)], den_sp.at[pl.ds(r0, NPT)])
        plsc.subcore_barrier()

        def issue_idx(q, sidx, didx):
            pltpu.async_copy(src_h.at[wid, pl.ds(q, 2)], sidx, sem_i)
            pltpu.async_copy(dst_h.at[wid, pl.ds(q, 2)], didx, sem_i)

        def wait_idx(q, sidx, didx):
            pltpu.make_async_copy(src_h.at[wid, pl.ds(q, 2)], sidx,
                                  sem_i).wait()
            pltpu.make_async_copy(dst_h.at[wid, pl.ds(q, 2)], didx,
                                  sem_i).wait()

        def issue_gathers(sidx_row, didx_row, rows, elv, erv, sem):
            pltpu.async_copy(fsrc_h.at[sidx_row], rows, sem)
            pltpu.async_copy(el_h.at[sidx_row], elv, sem)
            pltpu.async_copy(er_h.at[didx_row], erv, sem)

        def wait_gathers(sidx_row, didx_row, rows, elv, erv, sem):
            pltpu.make_async_copy(fsrc_h.at[sidx_row], rows, sem).wait()
            pltpu.make_async_copy(el_h.at[sidx_row], elv, sem).wait()
            pltpu.make_async_copy(er_h.at[didx_row], erv, sem).wait()

        def compute_scatter(j, didx_row, rows, elv, erv):
            base = j * CH

            def edge(e, carry2):
                x = elv[e, :] + erv[e, :]
                x = jnp.where(x >= 0.0, x, 0.2 * x)
                ee = jnp.exp(x)
                ee = jnp.where(base + e < EPW, ee,
                               jnp.zeros((OUT,), jnp.float32))
                plsc.store_scatter(
                    eev, [jnp.full((OUT,), e, jnp.int32), lane], ee,
                    mask=lanemask)
                for h in range(HEADS):
                    w = jnp.take_along_axis(
                        ee, jnp.full((OUT,), h, jnp.int32), axis=0,
                        mode="promise_in_bounds")
                    rows[e, pl.ds(h * OUT, OUT)] = \
                        rows[e, pl.ds(h * OUT, OUT)] * w
                return carry2

            lax.fori_loop(0, CH, edge, 0)
            pltpu.sync_copy(eev, den_sp.at[didx_row], add=True)
            pltpu.sync_copy(rows, acc_sp.at[didx_row], add=True)

        pltpu.sync_copy(src_h.at[wid, pl.ds(0, 2)], sidx_a)
        pltpu.sync_copy(dst_h.at[wid, pl.ds(0, 2)], didx_a)
        issue_gathers(sidx_a.at[0], didx_a.at[0], rows_a, el_a, er_a, sem_a)

        def quad(k, carry):
            j = 4 * k
            qb = jnp.minimum(j + 2, NCHUNK - 2)
            qa = jnp.minimum(j + 4, NCHUNK - 2)
            issue_idx(qb, sidx_b, didx_b)
            wait_gathers(sidx_a.at[0], didx_a.at[0], rows_a, el_a, er_a, sem_a)
            issue_gathers(sidx_a.at[1], didx_a.at[1], rows_b, el_b, er_b,
                          sem_b)
            compute_scatter(j, didx_a.at[0], rows_a, el_a, er_a)
            wait_gathers(sidx_a.at[1], didx_a.at[1], rows_b, el_b, er_b,
                         sem_b)
            wait_idx(qb, sidx_b, didx_b)
            issue_gathers(sidx_b.at[0], didx_b.at[0], rows_a, el_a, er_a,
                          sem_a)
            compute_scatter(j + 1, didx_a.at[1], rows_b, el_b, er_b)
            issue_idx(qa, sidx_a, didx_a)
            wait_gathers(sidx_b.at[0], didx_b.at[0], rows_a, el_a, er_a,
                         sem_a)
            issue_gathers(sidx_b.at[1], didx_b.at[1], rows_b, el_b, er_b,
                          sem_b)
            compute_scatter(j + 2, didx_b.at[0], rows_a, el_a, er_a)
            wait_gathers(sidx_b.at[1], didx_b.at[1], rows_b, el_b, er_b,
                         sem_b)
            wait_idx(qa, sidx_a, didx_a)
            issue_gathers(sidx_a.at[0], didx_a.at[0], rows_a, el_a, er_a,
                          sem_a)
            compute_scatter(j + 3, didx_b.at[1], rows_b, el_b, er_b)
            return carry

        lax.fori_loop(0, NCHUNK // 4, quad, 0)
        wait_gathers(sidx_a.at[0], didx_a.at[0], rows_a, el_a, er_a, sem_a)
        plsc.subcore_barrier()
        pltpu.sync_copy(acc_sp.at[pl.ds(r0, NPT)], acc_o.at[c, pl.ds(r0, NPT)])
        pltpu.sync_copy(den_sp.at[pl.ds(r0, NPT)], den_o.at[c, pl.ds(r0, NPT)])
        plsc.subcore_barrier()


def _sc_edge(fsrcs, els, ers, srcs, dsts, zacc, zden):
    mesh = plsc.VectorSubcoreMesh(
        core_axis_name="c", subcore_axis_name="s",
        num_cores=NC, num_subcores=NS)
    f = pl.kernel(
        _sc_edge_body,
        out_type=[jax.ShapeDtypeStruct((NC, NPAD, D), jnp.float32)] * 3
        + [jax.ShapeDtypeStruct((NC, NPAD, HEADS), jnp.float32)] * 3,
        mesh=mesh,
        compiler_params=pltpu.CompilerParams(
            use_tc_tiling_on_sc=False, needs_layout_passes=False),
        scratch_types=[
            pltpu.VMEM((2, CH), jnp.int32),
            pltpu.VMEM((2, CH), jnp.int32),
            pltpu.VMEM((2, CH), jnp.int32),
            pltpu.VMEM((2, CH), jnp.int32),
            pltpu.VMEM((CH, D), jnp.float32),
            pltpu.VMEM((CH, D), jnp.float32),
            pltpu.VMEM((CH, OUT), jnp.float32),
            pltpu.VMEM((CH, OUT), jnp.float32),
            pltpu.VMEM((CH, OUT), jnp.float32),
            pltpu.VMEM((CH, OUT), jnp.float32),
            pltpu.VMEM((CH, HEADS), jnp.float32),
            pltpu.VMEM_SHARED((NPAD, D), jnp.float32),
            pltpu.VMEM_SHARED((NPAD, HEADS), jnp.float32),
            pltpu.SemaphoreType.DMA,
            pltpu.SemaphoreType.DMA,
            pltpu.SemaphoreType.DMA,
        ])
    return f(*fsrcs, *els, *ers, *srcs, *dsts, zacc, zden)


def _pad_edges(ei):
    pad = jnp.broadcast_to(
        (jnp.arange(EPAD - EPW, dtype=jnp.int32) % N)[None],
        (NW, EPAD - EPW))
    def prep(v):
        v2 = v.astype(jnp.int32).reshape(NW, EPW)
        return jnp.concatenate([v2, pad], axis=1).reshape(NW, NCHUNK, CH)
    return prep(ei[0]), prep(ei[1])


def _expand_attn(a):
    eye = jnp.eye(HEADS, dtype=jnp.float32)
    return (a[:, :, None] * eye[:, None, :]).reshape(HEADS * OUT, HEADS)


def kernel(h_emo, h_cau, h_pair, edge_index_emo, edge_index_cau,
           edge_index_pair, doc_len, params):
    feats = (h_emo, h_cau, h_pair)
    eis = (edge_index_emo, edge_index_cau, edge_index_pair)
    b16 = (jnp.eye(HEADS, dtype=jnp.float32)[:, :, None]
           * jnp.ones((1, 1, OUT), jnp.float32)).reshape(HEADS, D)

    fsrcs, els, ers, srcs, dsts, ress = [], [], [], [], [], []
    p = params
    for i in range(3):
        a_l = _expand_attn(p['attn_l_%d' % i])
        a_r = _expand_attn(p['attn_r_%d' % i])
        bias = p['bias_%d' % i].reshape(1, D)
        fsrc, el16, er16, res = _tc1(
            feats[i], p['W_src_%d' % i], p['W_dst_%d' % i], p['W_res_%d' % i],
            a_l, a_r, bias)
        src3, dst3 = _pad_edges(eis[i])
        fsrcs.append(fsrc); els.append(el16); ers.append(er16)
        srcs.append(src3); dsts.append(dst3); ress.append(res)
    zacc = jnp.zeros((NPAD, D), jnp.float32)
    zden = jnp.zeros((NPAD, HEADS), jnp.float32)
    a0, a1, a2, n0, n1, n2 = _sc_edge(fsrcs, els, ers, srcs, dsts, zacc, zden)
    outs = []
    for i, (acc2, den2) in enumerate(((a0, n0), (a1, n1), (a2, n2))):
        outs.append(_tc2(acc2[0, :N], acc2[1, :N], den2[0, :N], den2[1, :N],
                         ress[i], b16))
    return tuple(outs)


# speedup vs baseline: 388.4822x; 1.5528x over previous
import functools

import jax
import jax.numpy as jnp
from jax import lax
from jax.experimental import pallas as pl
from jax.experimental.pallas import tpu as pltpu
from jax.experimental.pallas import tpu_sc as plsc

N = 10000
E = 320000
HEADS = 8
OUT = 16
D = 128
BN = 1000

NC = 2
NS = 16
NW = NC * NS
EPW = E // NW
CH = 128
NCHUNK = 80
EPAD = NCHUNK * CH
NPAD = 10112
NPT = NPAD // NS


def _tc1_body(x_ref, ws_ref, wd_ref, wr_ref, al_ref, ar_ref, b_ref,
              fs_ref, el_ref, er_ref, res_ref):
    x = x_ref[...]
    fs = jnp.dot(x, ws_ref[...], preferred_element_type=jnp.float32)
    fd = jnp.dot(x, wd_ref[...], preferred_element_type=jnp.float32)
    fs_ref[...] = fs
    el = jnp.dot(fs, al_ref[...], preferred_element_type=jnp.float32)
    er = jnp.dot(fd, ar_ref[...], preferred_element_type=jnp.float32)
    el_ref[...] = jnp.concatenate(
        [el, jnp.full((BN, HEADS), -1e30, jnp.float32)], axis=1)
    er_ref[...] = jnp.concatenate(
        [er, jnp.zeros((BN, HEADS), jnp.float32)], axis=1)
    res_ref[...] = jnp.dot(x, wr_ref[...], preferred_element_type=jnp.float32) \
        + b_ref[...]


def _tc1(x, w_src, w_dst, w_res, a_l, a_r, bias):
    grid = N // BN
    wspec = pl.BlockSpec((D, D), lambda i: (0, 0))
    return pl.pallas_call(
        _tc1_body,
        grid=(grid,),
        in_specs=[
            pl.BlockSpec((BN, D), lambda i: (i, 0)),
            wspec, wspec, wspec,
            pl.BlockSpec((D, HEADS), lambda i: (0, 0)),
            pl.BlockSpec((D, HEADS), lambda i: (0, 0)),
            pl.BlockSpec((1, D), lambda i: (0, 0)),
        ],
        out_specs=[
            pl.BlockSpec((BN, D), lambda i: (i, 0)),
            pl.BlockSpec((BN, OUT), lambda i: (i, 0)),
            pl.BlockSpec((BN, OUT), lambda i: (i, 0)),
            pl.BlockSpec((BN, D), lambda i: (i, 0)),
        ],
        out_shape=[
            jax.ShapeDtypeStruct((N, D), jnp.float32),
            jax.ShapeDtypeStruct((N, OUT), jnp.float32),
            jax.ShapeDtypeStruct((N, OUT), jnp.float32),
            jax.ShapeDtypeStruct((N, D), jnp.float32),
        ],
    )(x, w_src, w_dst, w_res, a_l, a_r, bias)


def _tc2_body(acca_ref, accb_ref, dena_ref, denb_ref, res_ref, b16_ref, o_ref):
    den = dena_ref[...] + denb_ref[...]
    denw = jnp.dot(den, b16_ref[...],
                   preferred_element_type=jnp.float32)
    safe = jnp.where(denw > 0.0, denw, 1.0)
    y = (acca_ref[...] + accb_ref[...]) / safe + res_ref[...]
    o_ref[...] = jnp.where(y > 0.0, y, jnp.exp(jnp.minimum(y, 0.0)) - 1.0)


def _tc2(acc_a, acc_b, den_a, den_b, res, b16):
    grid = N // BN
    return pl.pallas_call(
        _tc2_body,
        grid=(grid,),
        in_specs=[
            pl.BlockSpec((BN, D), lambda i: (i, 0)),
            pl.BlockSpec((BN, D), lambda i: (i, 0)),
            pl.BlockSpec((BN, HEADS), lambda i: (i, 0)),
            pl.BlockSpec((BN, HEADS), lambda i: (i, 0)),
            pl.BlockSpec((BN, D), lambda i: (i, 0)),
            pl.BlockSpec((HEADS, D), lambda i: (0, 0)),
        ],
        out_specs=pl.BlockSpec((BN, D), lambda i: (i, 0)),
        out_shape=jax.ShapeDtypeStruct((N, D), jnp.float32),
    )(acc_a, acc_b, den_a, den_b, res, b16)


def _sc_edge_body(*refs):
    (f0, f1, f2, l0, l1, l2, r0_, r1_, r2_, s0, s1, s2, d0, d1, d2,
     zacc_h, zden_h,
     a0, a1, a2, n0, n1, n2,
     sidx_a, sidx_b, didx_a, didx_b,
     rows_a, rows_b, el_a, el_b, er_a, er_b, eev,
     acc_sp, den_sp, sem_a, sem_b, sem_i) = refs
    c = lax.axis_index("c")
    s = lax.axis_index("s")
    wid = c * NS + s
    r0 = s * NPT
    lane = lax.iota(jnp.int32, OUT)
    lanemask = lane < HEADS

    for (fsrc_h, el_h, er_h, src_h, dst_h, acc_o, den_o) in (
            (f0, l0, r0_, s0, d0, a0, n0),
            (f1, l1, r1_, s1, d1, a1, n1),
            (f2, l2, r2_, s2, d2, a2, n2)):
        pltpu.sync_copy(zacc_h.at[pl.ds(r0, NPT)], acc_sp.at[pl.ds(r0, NPT)])
        pltpu.sync_copy(zden_h.at[pl.ds(r0, NPT)], den_sp.at[pl.ds(r0, NPT)])
        plsc.subcore_barrier()

        def issue_idx(q, sidx, didx):
            pltpu.async_copy(src_h.at[wid, pl.ds(q, 2)], sidx, sem_i)
            pltpu.async_copy(dst_h.at[wid, pl.ds(q, 2)], didx, sem_i)

        def wait_idx(q, sidx, didx):
            pltpu.make_async_copy(src_h.at[wid, pl.ds(q, 2)], sidx,
                                  sem_i).wait()
            pltpu.make_async_copy(dst_h.at[wid, pl.ds(q, 2)], didx,
                                  sem_i).wait()

        def issue_gathers(sidx_row, didx_row, rows, elv, erv, sem):
            pltpu.async_copy(fsrc_h.at[sidx_row], rows, sem)
            pltpu.async_copy(el_h.at[sidx_row], elv, sem)
            pltpu.async_copy(er_h.at[didx_row], erv, sem)

        def wait_gathers(sidx_row, didx_row, rows, elv, erv, sem):
            pltpu.make_async_copy(fsrc_h.at[sidx_row], rows, sem).wait()
            pltpu.make_async_copy(el_h.at[sidx_row], elv, sem).wait()
            pltpu.make_async_copy(er_h.at[didx_row], erv, sem).wait()

        def compute_scatter(j, didx_row, rows, elv, erv):
            base = j * CH

            @plsc.parallel_loop(0, CH, step=1, unroll=4)
            def edge(e):
                x = elv[e, :] + erv[e, :]
                x = jnp.where(x >= 0.0, x, 0.2 * x)
                ee = jnp.exp(x)
                ee = jnp.where(base + e < EPW, ee,
                               jnp.zeros((OUT,), jnp.float32))
                plsc.store_scatter(
                    eev, [jnp.full((OUT,), e, jnp.int32), lane], ee,
                    mask=lanemask)
                for h in range(HEADS):
                    w = jnp.take_along_axis(
                        ee, jnp.full((OUT,), h, jnp.int32), axis=0,
                        mode="promise_in_bounds")
                    rows[e, pl.ds(h * OUT, OUT)] = \
                        rows[e, pl.ds(h * OUT, OUT)] * w
            pltpu.sync_copy(eev, den_sp.at[didx_row], add=True)
            pltpu.sync_copy(rows, acc_sp.at[didx_row], add=True)

        pltpu.sync_copy(src_h.at[wid, pl.ds(0, 2)], sidx_a)
        pltpu.sync_copy(dst_h.at[wid, pl.ds(0, 2)], didx_a)
        issue_gathers(sidx_a.at[0], didx_a.at[0], rows_a, el_a, er_a, sem_a)

        def quad(k, carry):
            j = 4 * k
            qb = jnp.minimum(j + 2, NCHUNK - 2)
            qa = jnp.minimum(j + 4, NCHUNK - 2)
            issue_idx(qb, sidx_b, didx_b)
            wait_gathers(sidx_a.at[0], didx_a.at[0], rows_a, el_a, er_a, sem_a)
            issue_gathers(sidx_a.at[1], didx_a.at[1], rows_b, el_b, er_b,
                          sem_b)
            compute_scatter(j, didx_a.at[0], rows_a, el_a, er_a)
            wait_gathers(sidx_a.at[1], didx_a.at[1], rows_b, el_b, er_b,
                         sem_b)
            wait_idx(qb, sidx_b, didx_b)
            issue_gathers(sidx_b.at[0], didx_b.at[0], rows_a, el_a, er_a,
                          sem_a)
            compute_scatter(j + 1, didx_a.at[1], rows_b, el_b, er_b)
            issue_idx(qa, sidx_a, didx_a)
            wait_gathers(sidx_b.at[0], didx_b.at[0], rows_a, el_a, er_a,
                         sem_a)
            issue_gathers(sidx_b.at[1], didx_b.at[1], rows_b, el_b, er_b,
                          sem_b)
            compute_scatter(j + 2, didx_b.at[0], rows_a, el_a, er_a)
            wait_gathers(sidx_b.at[1], didx_b.at[1], rows_b, el_b, er_b,
                         sem_b)
            wait_idx(qa, sidx_a, didx_a)
            issue_gathers(sidx_a.at[0], didx_a.at[0], rows_a, el_a, er_a,
                          sem_a)
            compute_scatter(j + 3, didx_b.at[1], rows_b, el_b, er_b)
            return carry

        lax.fori_loop(0, NCHUNK // 4, quad, 0)
        wait_gathers(sidx_a.at[0], didx_a.at[0], rows_a, el_a, er_a, sem_a)
        plsc.subcore_barrier()
        pltpu.sync_copy(acc_sp.at[pl.ds(r0, NPT)], acc_o.at[c, pl.ds(r0, NPT)])
        pltpu.sync_copy(den_sp.at[pl.ds(r0, NPT)], den_o.at[c, pl.ds(r0, NPT)])
        plsc.subcore_barrier()


def _sc_edge(fsrcs, els, ers, srcs, dsts, zacc, zden):
    mesh = plsc.VectorSubcoreMesh(
        core_axis_name="c", subcore_axis_name="s",
        num_cores=NC, num_subcores=NS)
    f = pl.kernel(
        _sc_edge_body,
        out_type=[jax.ShapeDtypeStruct((NC, NPAD, D), jnp.float32)] * 3
        + [jax.ShapeDtypeStruct((NC, NPAD, HEADS), jnp.float32)] * 3,
        mesh=mesh,
        compiler_params=pltpu.CompilerParams(
            use_tc_tiling_on_sc=False, needs_layout_passes=False),
        scratch_types=[
            pltpu.VMEM((2, CH), jnp.int32),
            pltpu.VMEM((2, CH), jnp.int32),
            pltpu.VMEM((2, CH), jnp.int32),
            pltpu.VMEM((2, CH), jnp.int32),
            pltpu.VMEM((CH, D), jnp.float32),
            pltpu.VMEM((CH, D), jnp.float32),
            pltpu.VMEM((CH, OUT), jnp.float32),
            pltpu.VMEM((CH, OUT), jnp.float32),
            pltpu.VMEM((CH, OUT), jnp.float32),
            pltpu.VMEM((CH, OUT), jnp.float32),
            pltpu.VMEM((CH, HEADS), jnp.float32),
            pltpu.VMEM_SHARED((NPAD, D), jnp.float32),
            pltpu.VMEM_SHARED((NPAD, HEADS), jnp.float32),
            pltpu.SemaphoreType.DMA,
            pltpu.SemaphoreType.DMA,
            pltpu.SemaphoreType.DMA,
        ])
    return f(*fsrcs, *els, *ers, *srcs, *dsts, zacc, zden)


def _pad_edges(ei):
    pad = jnp.broadcast_to(
        (jnp.arange(EPAD - EPW, dtype=jnp.int32) % N)[None],
        (NW, EPAD - EPW))
    def prep(v):
        v2 = v.astype(jnp.int32).reshape(NW, EPW)
        return jnp.concatenate([v2, pad], axis=1).reshape(NW, NCHUNK, CH)
    return prep(ei[0]), prep(ei[1])


def _expand_attn(a):
    eye = jnp.eye(HEADS, dtype=jnp.float32)
    return (a[:, :, None] * eye[:, None, :]).reshape(HEADS * OUT, HEADS)


def kernel(h_emo, h_cau, h_pair, edge_index_emo, edge_index_cau,
           edge_index_pair, doc_len, params):
    feats = (h_emo, h_cau, h_pair)
    eis = (edge_index_emo, edge_index_cau, edge_index_pair)
    b16 = (jnp.eye(HEADS, dtype=jnp.float32)[:, :, None]
           * jnp.ones((1, 1, OUT), jnp.float32)).reshape(HEADS, D)

    fsrcs, els, ers, srcs, dsts, ress = [], [], [], [], [], []
    p = params
    for i in range(3):
        a_l = _expand_attn(p['attn_l_%d' % i])
        a_r = _expand_attn(p['attn_r_%d' % i])
        bias = p['bias_%d' % i].reshape(1, D)
        fsrc, el16, er16, res = _tc1(
            feats[i], p['W_src_%d' % i], p['W_dst_%d' % i], p['W_res_%d' % i],
            a_l, a_r, bias)
        src3, dst3 = _pad_edges(eis[i])
        fsrcs.append(fsrc); els.append(el16); ers.append(er16)
        srcs.append(src3); dsts.append(dst3); ress.append(res)
    zacc = jnp.zeros((NPAD, D), jnp.float32)
    zden = jnp.zeros((NPAD, HEADS), jnp.float32)
    a0, a1, a2, n0, n1, n2 = _sc_edge(fsrcs, els, ers, srcs, dsts, zacc, zden)
    outs = []
    for i, (acc2, den2) in enumerate(((a0, n0), (a1, n1), (a2, n2))):
        outs.append(_tc2(acc2[0, :N], acc2[1, :N], den2[0, :N], den2[1, :N],
                         ress[i], b16))
    return tuple(outs)
